# baseline (speedup 1.0000x reference)
import functools

import jax
import jax.numpy as jnp
from jax import lax
from jax.experimental import pallas as pl
from jax.experimental.pallas import tpu as pltpu
from jax.experimental.pallas import tpu_sc as plsc

N = 10000
F = 128
E = 320000
NC = 2
NS = 16
NW = NC * NS
CHUNK = 128
N_PAD = 10016
E_PAD = 327680
TOTAL_CHUNKS = E_PAD // CHUNK
NCH = TOTAL_CHUNKS // NW
BLK = 16
NBLK = N_PAD // BLK
BASE_BLKS = NBLK // NS
BLK_EXTRA = NBLK - BASE_BLKS * NS
DW = 16

_mesh = plsc.VectorSubcoreMesh(core_axis_name="c", subcore_axis_name="s")


def _row_blocks(s):
    return BASE_BLKS + jnp.where(s < BLK_EXTRA, 1, 0)


def _load_indices(src2_hbm, dst2_hbm, si_v, di_v, wid):
    ch0 = wid * NCH
    pltpu.sync_copy(src2_hbm.at[pl.ds(ch0, NCH)], si_v)
    pltpu.sync_copy(dst2_hbm.at[pl.ds(ch0, NCH)], di_v)


@functools.partial(
    pl.kernel,
    out_type=jax.ShapeDtypeStruct((NC, 2, N_PAD, DW), jnp.float32),
    mesh=_mesh,
    compiler_params=pltpu.CompilerParams(use_tc_tiling_on_sc=False),
    scratch_types=[
        pltpu.VMEM((NCH, CHUNK), jnp.int32),
        pltpu.VMEM((NCH, CHUNK), jnp.int32),
        pltpu.VMEM((CHUNK, DW), jnp.float32),
        pltpu.VMEM((BLK, DW), jnp.float32),
        pltpu.VMEM_SHARED((N_PAD, DW), jnp.float32),
        pltpu.VMEM_SHARED((N_PAD, DW), jnp.float32),
        pltpu.SemaphoreType.DMA,
    ],
)
def _sc_degrees(src2_hbm, dst2_hbm, out_hbm, si_v, di_v, ones_v, zsl_v,
                acca_sh, accb_sh, sem):
    c = lax.axis_index("c")
    s = lax.axis_index("s")
    wid = s * NC + c

    def fill(i, carry):
        ones_v[i, pl.ds(0, 16)] = jnp.full((16,), 1.0, jnp.float32)
        return carry

    lax.fori_loop(0, CHUNK, fill, 0)

    def zfill(i, carry):
        zsl_v[i, pl.ds(0, 16)] = jnp.zeros((16,), jnp.float32)
        return carry

    lax.fori_loop(0, BLK, zfill, 0)

    _load_indices(src2_hbm, dst2_hbm, si_v, di_v, wid)

    def zcopy(i, carry):
        row = (s + i * NS) * BLK
        pltpu.sync_copy(zsl_v, acca_sh.at[pl.ds(row, BLK)])
        pltpu.sync_copy(zsl_v, accb_sh.at[pl.ds(row, BLK)])
        return carry

    lax.fori_loop(0, _row_blocks(s), zcopy, 0)
    plsc.subcore_barrier()

    def body(g, carry):
        pltpu.async_copy(ones_v, acca_sh.at[si_v.at[g]], sem, add=True)
        pltpu.async_copy(ones_v, accb_sh.at[di_v.at[g]], sem, add=True)
        return carry

    lax.fori_loop(0, NCH, body, 0)

    def drain(i, carry):
        pltpu.make_async_copy(out_hbm.at[c, 0, pl.ds(0, CHUNK)], ones_v, sem).wait()
        return carry

    lax.fori_loop(0, 2 * NCH, drain, 0)
    plsc.subcore_barrier()

    def wcopy(i, carry):
        row = (s + i * NS) * BLK
        pltpu.sync_copy(acca_sh.at[pl.ds(row, BLK)], out_hbm.at[c, 0, pl.ds(row, BLK)])
        pltpu.sync_copy(accb_sh.at[pl.ds(row, BLK)], out_hbm.at[c, 1, pl.ds(row, BLK)])
        return carry

    lax.fori_loop(0, _row_blocks(s), wcopy, 0)


@functools.partial(
    pl.kernel,
    out_type=jax.ShapeDtypeStruct((NC, N_PAD, F), jnp.float32),
    mesh=_mesh,
    scratch_types=[
        pltpu.VMEM((CHUNK,), jnp.int32),
        pltpu.VMEM((CHUNK,), jnp.int32),
        pltpu.VMEM((CHUNK, F), jnp.float32),
        pltpu.VMEM((BLK, F), jnp.float32),
        pltpu.VMEM_SHARED((N_PAD, F), jnp.float32),
        pltpu.SemaphoreType.DMA,
    ],
)
def _sc_aggregate(hs_hbm, src1_hbm, dst1_hbm, out_hbm, si_v, di_v, msg_v, zsl_v,
                  acc_sh, sem):
    c = lax.axis_index("c")
    s = lax.axis_index("s")
    wid = s * NC + c
    ebase = wid * NCH * CHUNK

    def zfill(i, carry):
        def zcol(j, c2):
            zsl_v[i, pl.ds(j * 16, 16)] = jnp.zeros((16,), jnp.float32)
            return c2

        return lax.fori_loop(0, F // 16, zcol, carry)

    lax.fori_loop(0, BLK, zfill, 0)

    def zcopy(i, carry):
        row = (s + i * NS) * BLK
        pltpu.sync_copy(zsl_v, acc_sh.at[pl.ds(row, BLK)])
        return carry

    lax.fori_loop(0, _row_blocks(s), zcopy, 0)
    plsc.subcore_barrier()

    def body(g, carry):
        base = ebase + g * CHUNK
        pltpu.sync_copy(src1_hbm.at[pl.ds(base, CHUNK)], si_v)
        pltpu.sync_copy(dst1_hbm.at[pl.ds(base, CHUNK)], di_v)
        pltpu.async_copy(hs_hbm.at[si_v], msg_v, sem).wait()
        pltpu.sync_copy(msg_v, acc_sh.at[di_v], add=True)
        return carry

    lax.fori_loop(0, NCH, body, 0)
    plsc.subcore_barrier()

    def wcopy(i, carry):
        row = (s + i * NS) * BLK
        pltpu.sync_copy(acc_sh.at[pl.ds(row, BLK)], out_hbm.at[c, pl.ds(row, BLK)])
        return carry

    lax.fori_loop(0, _row_blocks(s), wcopy, 0)


def _norm_body(dp_ref, nm_ref):
    deg = dp_ref[0] + dp_ref[1]
    nm = lax.rsqrt(jnp.maximum(deg[:, :, 0:1], 1.0))
    row = lax.broadcasted_iota(jnp.int32, (1, N_PAD, 1), 1)
    nm_ref[...] = jnp.where(row < N, nm, 0.0)


def _scale_body(x_ref, ns_ref, hs_ref):
    hs_ref[...] = x_ref[...] * ns_ref[...]


def _bn_relu(p_ref, nd_ref, w_ref, b_ref, g_ref, be_ref):
    agg = (p_ref[0] + p_ref[1]) * nd_ref[...]
    hp = jnp.dot(agg, w_ref[...], preferred_element_type=jnp.float32) + b_ref[...]
    hpn = hp[0:N]
    mu = jnp.mean(hpn, axis=0, keepdims=True)
    var = jnp.mean((hpn - mu) ** 2, axis=0, keepdims=True)
    return jnp.maximum((hp - mu) * lax.rsqrt(var + 1e-5) * g_ref[...] + be_ref[...], 0.0)


def _dense_body(p_ref, nd_ref, ns_ref, w_ref, b_ref, g_ref, be_ref, o_ref):
    h = _bn_relu(p_ref, nd_ref, w_ref, b_ref, g_ref, be_ref)
    o_ref[...] = h * ns_ref[...]


def _head_body(p_ref, nd_ref, w_ref, b_ref, g_ref, be_ref, wc_ref, bc_ref, o_ref):
    h = _bn_relu(p_ref, nd_ref, w_ref, b_ref, g_ref, be_ref)
    hg = jnp.mean(h[0:N], axis=0, keepdims=True)
    o_ref[...] = jnp.dot(hg, wc_ref[...], preferred_element_type=jnp.float32) + bc_ref[...]


def kernel(x, edge_index, W1, b1, g1, be1, W2, b2, g2, be2, Wc, bc):
    f32 = jnp.float32
    i32 = jnp.int32
    pad = jnp.full((E_PAD - E,), N, i32)
    src2 = jnp.concatenate([edge_index[0].astype(i32), pad]).reshape(TOTAL_CHUNKS, CHUNK)
    dst2 = jnp.concatenate([edge_index[1].astype(i32), pad]).reshape(TOTAL_CHUNKS, CHUNK)
    x_pad = jnp.concatenate([x, jnp.zeros((N_PAD - N, F), f32)])

    dp = _sc_degrees(src2, dst2)

    nm = pl.pallas_call(
        _norm_body, out_shape=jax.ShapeDtypeStruct((2, N_PAD, 1), f32)
    )(dp)
    ns_col = nm[0]
    nd_col = nm[1]

    hs = pl.pallas_call(
        _scale_body, out_shape=jax.ShapeDtypeStruct((N_PAD, F), f32)
    )(x_pad, ns_col)

    src1 = src2.reshape(E_PAD)
    dst1 = dst2.reshape(E_PAD)
    p1 = _sc_aggregate(hs, src1, dst1)

    h1s = pl.pallas_call(
        _dense_body, out_shape=jax.ShapeDtypeStruct((N_PAD, F), f32)
    )(p1, nd_col, ns_col, W1, b1.reshape(1, F), g1.reshape(1, F), be1.reshape(1, F))

    p2 = _sc_aggregate(h1s, src1, dst1)

    out = pl.pallas_call(
        _head_body, out_shape=jax.ShapeDtypeStruct((1, 2), f32)
    )(p2, nd_col, W2, b2.reshape(1, F), g2.reshape(1, F), be2.reshape(1, F),
      Wc, bc.reshape(1, 2))
    return out


# speedup vs baseline: 2.0890x; 2.0890x over previous
import functools

import jax
import jax.numpy as jnp
from jax import lax
from jax.experimental import pallas as pl
from jax.experimental.pallas import tpu as pltpu
from jax.experimental.pallas import tpu_sc as plsc

N = 10000
F = 128
E = 320000
NC = 2
NS = 16
NW = NC * NS
CHUNK = 128
TOTAL_CHUNKS = E // CHUNK
BASE_CHUNKS = TOTAL_CHUNKS // NW
EXTRA = TOTAL_CHUNKS - BASE_CHUNKS * NW
BLK = 16
NBLK = N // BLK
BASE_BLKS = NBLK // NS
BLK_EXTRA = NBLK - BASE_BLKS * NS
DW = 16

_mesh = plsc.VectorSubcoreMesh(core_axis_name="c", subcore_axis_name="s")


def _edge_span(wid):
    nch = BASE_CHUNKS + jnp.where(wid < EXTRA, 1, 0)
    ch0 = wid * BASE_CHUNKS + jnp.minimum(wid, EXTRA)
    return ch0, nch


def _row_blocks(s):
    return BASE_BLKS + jnp.where(s < BLK_EXTRA, 1, 0)


@functools.partial(
    pl.kernel,
    out_type=jax.ShapeDtypeStruct((NC, 2, N, DW), jnp.float32),
    mesh=_mesh,
    compiler_params=pltpu.CompilerParams(use_tc_tiling_on_sc=False),
    scratch_types=[
        pltpu.VMEM((CHUNK,), jnp.int32),
        pltpu.VMEM((CHUNK,), jnp.int32),
        pltpu.VMEM((CHUNK, DW), jnp.float32),
        pltpu.VMEM((BLK, DW), jnp.float32),
        pltpu.VMEM_SHARED((N, DW), jnp.float32),
        pltpu.VMEM_SHARED((N, DW), jnp.float32),
    ],
)
def _sc_degrees(src_hbm, dst_hbm, out_hbm, si_v, di_v, ones_v, zsl_v, acca_sh, accb_sh):
    c = lax.axis_index("c")
    s = lax.axis_index("s")
    wid = s * NC + c

    def fill(i, carry):
        ones_v[i, pl.ds(0, 16)] = jnp.full((16,), 1.0, jnp.float32)
        return carry

    lax.fori_loop(0, CHUNK, fill, 0)

    def zfill(i, carry):
        zsl_v[i, pl.ds(0, 16)] = jnp.zeros((16,), jnp.float32)
        return carry

    lax.fori_loop(0, BLK, zfill, 0)

    def zcopy(i, carry):
        row = (s + i * NS) * BLK
        pltpu.sync_copy(zsl_v, acca_sh.at[pl.ds(row, BLK)])
        pltpu.sync_copy(zsl_v, accb_sh.at[pl.ds(row, BLK)])
        return carry

    lax.fori_loop(0, _row_blocks(s), zcopy, 0)
    plsc.subcore_barrier()

    ch0, nch = _edge_span(wid)

    def body(g, carry):
        base = (ch0 + g) * CHUNK
        pltpu.sync_copy(src_hbm.at[pl.ds(base, CHUNK)], si_v)
        pltpu.sync_copy(dst_hbm.at[pl.ds(base, CHUNK)], di_v)
        pltpu.sync_copy(ones_v, acca_sh.at[si_v], add=True)
        pltpu.sync_copy(ones_v, accb_sh.at[di_v], add=True)
        return carry

    lax.fori_loop(0, nch, body, 0)
    plsc.subcore_barrier()

    def wcopy(i, carry):
        row = (s + i * NS) * BLK
        pltpu.sync_copy(acca_sh.at[pl.ds(row, BLK)], out_hbm.at[c, 0, pl.ds(row, BLK)])
        pltpu.sync_copy(accb_sh.at[pl.ds(row, BLK)], out_hbm.at[c, 1, pl.ds(row, BLK)])
        return carry

    lax.fori_loop(0, _row_blocks(s), wcopy, 0)


@functools.partial(
    pl.kernel,
    out_type=jax.ShapeDtypeStruct((NC, N, F), jnp.float32),
    mesh=_mesh,
    scratch_types=[
        pltpu.VMEM((CHUNK,), jnp.int32),
        pltpu.VMEM((CHUNK,), jnp.int32),
        pltpu.VMEM((CHUNK, F), jnp.float32),
        pltpu.VMEM((BLK, F), jnp.float32),
        pltpu.VMEM_SHARED((N, F), jnp.float32),
        pltpu.SemaphoreType.DMA,
    ],
)
def _sc_aggregate(hs_hbm, src_hbm, dst_hbm, out_hbm, si_v, di_v, msg_v, zsl_v, acc_sh, sem):
    c = lax.axis_index("c")
    s = lax.axis_index("s")
    wid = s * NC + c

    def zfill(i, carry):
        def zcol(j, c2):
            zsl_v[i, pl.ds(j * 16, 16)] = jnp.zeros((16,), jnp.float32)
            return c2

        return lax.fori_loop(0, F // 16, zcol, carry)

    lax.fori_loop(0, BLK, zfill, 0)

    def zcopy(i, carry):
        row = (s + i * NS) * BLK
        pltpu.sync_copy(zsl_v, acc_sh.at[pl.ds(row, BLK)])
        return carry

    lax.fori_loop(0, _row_blocks(s), zcopy, 0)
    plsc.subcore_barrier()

    ch0, nch = _edge_span(wid)

    def body(g, carry):
        base = (ch0 + g) * CHUNK
        pltpu.sync_copy(src_hbm.at[pl.ds(base, CHUNK)], si_v)
        pltpu.sync_copy(dst_hbm.at[pl.ds(base, CHUNK)], di_v)
        pltpu.async_copy(hs_hbm.at[si_v], msg_v, sem).wait()
        pltpu.sync_copy(msg_v, acc_sh.at[di_v], add=True)
        return carry

    lax.fori_loop(0, nch, body, 0)
    plsc.subcore_barrier()

    def wcopy(i, carry):
        row = (s + i * NS) * BLK
        pltpu.sync_copy(acc_sh.at[pl.ds(row, BLK)], out_hbm.at[c, pl.ds(row, BLK)])
        return carry

    lax.fori_loop(0, _row_blocks(s), wcopy, 0)


def _norm_body(dp_ref, nm_ref):
    deg = dp_ref[0] + dp_ref[1]
    nm_ref[...] = lax.rsqrt(jnp.maximum(deg[:, :, 0:1], 1.0))


def _scale_body(x_ref, ns_ref, hs_ref):
    hs_ref[...] = x_ref[...] * ns_ref[...]


def _dense_body(p_ref, nd_ref, ns_ref, w_ref, b_ref, g_ref, be_ref, o_ref):
    agg = (p_ref[0] + p_ref[1]) * nd_ref[...]
    hp = jnp.dot(agg, w_ref[...], preferred_element_type=jnp.float32) + b_ref[...]
    mu = jnp.mean(hp, axis=0, keepdims=True)
    var = jnp.mean((hp - mu) ** 2, axis=0, keepdims=True)
    h = jnp.maximum((hp - mu) * lax.rsqrt(var + 1e-5) * g_ref[...] + be_ref[...], 0.0)
    o_ref[...] = h * ns_ref[...]


def _head_body(p_ref, nd_ref, w_ref, b_ref, g_ref, be_ref, wc_ref, bc_ref, o_ref):
    agg = (p_ref[0] + p_ref[1]) * nd_ref[...]
    hp = jnp.dot(agg, w_ref[...], preferred_element_type=jnp.float32) + b_ref[...]
    mu = jnp.mean(hp, axis=0, keepdims=True)
    var = jnp.mean((hp - mu) ** 2, axis=0, keepdims=True)
    h = jnp.maximum((hp - mu) * lax.rsqrt(var + 1e-5) * g_ref[...] + be_ref[...], 0.0)
    hg = jnp.mean(h, axis=0, keepdims=True)
    o_ref[...] = jnp.dot(hg, wc_ref[...], preferred_element_type=jnp.float32) + bc_ref[...]


def kernel(x, edge_index, W1, b1, g1, be1, W2, b2, g2, be2, Wc, bc):
    src = edge_index[0].astype(jnp.int32)
    dst = edge_index[1].astype(jnp.int32)
    f32 = jnp.float32

    dp = _sc_degrees(src, dst)

    nm = pl.pallas_call(
        _norm_body, out_shape=jax.ShapeDtypeStruct((2, N, 1), f32)
    )(dp)
    ns_col = nm[0]
    nd_col = nm[1]

    hs = pl.pallas_call(
        _scale_body, out_shape=jax.ShapeDtypeStruct((N, F), f32)
    )(x, ns_col)

    p1 = _sc_aggregate(hs, src, dst)

    h1s = pl.pallas_call(
        _dense_body, out_shape=jax.ShapeDtypeStruct((N, F), f32)
    )(p1, nd_col, ns_col, W1, b1.reshape(1, F), g1.reshape(1, F), be1.reshape(1, F))

    p2 = _sc_aggregate(h1s, src, dst)

    out = pl.pallas_call(
        _head_body, out_shape=jax.ShapeDtypeStruct((1, 2), f32)
    )(p2, nd_col, W2, b2.reshape(1, F), g2.reshape(1, F), be2.reshape(1, F),
      Wc, bc.reshape(1, 2))
    return out


# speedup vs baseline: 3.8340x; 1.8353x over previous
import functools

import jax
import jax.numpy as jnp
from jax import lax
from jax.experimental import pallas as pl
from jax.experimental.pallas import tpu as pltpu
from jax.experimental.pallas import tpu_sc as plsc

N = 10000
F = 128
E = 320000
NC = 2
NS = 16
NW = NC * NS
CHUNK = 128
N_PAD = 10016
E_PAD = 327680
TOTAL_CHUNKS = E_PAD // CHUNK
NCH = TOTAL_CHUNKS // NW
BLK = 16
NBLK = N_PAD // BLK
BASE_BLKS = NBLK // NS
BLK_EXTRA = NBLK - BASE_BLKS * NS
DW = 16

_mesh = plsc.VectorSubcoreMesh(core_axis_name="c", subcore_axis_name="s")


def _row_blocks(s):
    return BASE_BLKS + jnp.where(s < BLK_EXTRA, 1, 0)


def _load_indices(src2_hbm, dst2_hbm, si_v, di_v, wid):
    ch0 = wid * NCH
    pltpu.sync_copy(src2_hbm.at[pl.ds(ch0, NCH)], si_v)
    pltpu.sync_copy(dst2_hbm.at[pl.ds(ch0, NCH)], di_v)


@functools.partial(
    pl.kernel,
    out_type=jax.ShapeDtypeStruct((NC, 2, N_PAD, DW), jnp.float32),
    mesh=_mesh,
    compiler_params=pltpu.CompilerParams(use_tc_tiling_on_sc=False),
    scratch_types=[
        pltpu.VMEM((NCH, CHUNK), jnp.int32),
        pltpu.VMEM((NCH, CHUNK), jnp.int32),
        pltpu.VMEM((CHUNK, DW), jnp.float32),
        pltpu.VMEM((BLK, DW), jnp.float32),
        pltpu.VMEM_SHARED((N_PAD, DW), jnp.float32),
        pltpu.VMEM_SHARED((N_PAD, DW), jnp.float32),
        pltpu.SemaphoreType.DMA,
    ],
)
def _sc_degrees(src2_hbm, dst2_hbm, out_hbm, si_v, di_v, ones_v, zsl_v,
                acca_sh, accb_sh, sem):
    c = lax.axis_index("c")
    s = lax.axis_index("s")
    wid = s * NC + c

    def fill(i, carry):
        ones_v[i, pl.ds(0, 16)] = jnp.full((16,), 1.0, jnp.float32)
        return carry

    lax.fori_loop(0, CHUNK, fill, 0)

    def zfill(i, carry):
        zsl_v[i, pl.ds(0, 16)] = jnp.zeros((16,), jnp.float32)
        return carry

    lax.fori_loop(0, BLK, zfill, 0)

    _load_indices(src2_hbm, dst2_hbm, si_v, di_v, wid)

    def zcopy(i, carry):
        row = (s + i * NS) * BLK
        pltpu.sync_copy(zsl_v, acca_sh.at[pl.ds(row, BLK)])
        pltpu.sync_copy(zsl_v, accb_sh.at[pl.ds(row, BLK)])
        return carry

    lax.fori_loop(0, _row_blocks(s), zcopy, 0)
    plsc.subcore_barrier()

    def body(g, carry):
        pltpu.async_copy(ones_v, acca_sh.at[si_v.at[g]], sem, add=True)
        pltpu.async_copy(ones_v, accb_sh.at[di_v.at[g]], sem, add=True)
        return carry

    lax.fori_loop(0, NCH, body, 0)

    def drain(i, carry):
        pltpu.make_async_copy(out_hbm.at[c, 0, pl.ds(0, CHUNK)], ones_v, sem).wait()
        return carry

    lax.fori_loop(0, 2 * NCH, drain, 0)
    plsc.subcore_barrier()

    def wcopy(i, carry):
        row = (s + i * NS) * BLK
        pltpu.sync_copy(acca_sh.at[pl.ds(row, BLK)], out_hbm.at[c, 0, pl.ds(row, BLK)])
        pltpu.sync_copy(accb_sh.at[pl.ds(row, BLK)], out_hbm.at[c, 1, pl.ds(row, BLK)])
        return carry

    lax.fori_loop(0, _row_blocks(s), wcopy, 0)


@functools.partial(
    pl.kernel,
    out_type=jax.ShapeDtypeStruct((NC, N_PAD, F), jnp.float32),
    mesh=_mesh,
    scratch_types=[
        pltpu.VMEM((NCH, CHUNK), jnp.int32),
        pltpu.VMEM((2, CHUNK), jnp.int32),
        pltpu.VMEM((2, CHUNK, F), jnp.float32),
        pltpu.VMEM((BLK, F), jnp.float32),
        pltpu.VMEM_SHARED((N_PAD, F), jnp.float32),
        pltpu.SemaphoreType.DMA,
        pltpu.SemaphoreType.DMA,
        pltpu.SemaphoreType.DMA,
        pltpu.SemaphoreType.DMA,
    ],
)
def _sc_aggregate(hs_hbm, src2_hbm, dst2_hbm, out_hbm, si_v, di_v, msg_v, zsl_v,
                  acc_sh, gsem0, gsem1, isem0, isem1):
    c = lax.axis_index("c")
    s = lax.axis_index("s")
    wid = s * NC + c
    ch0 = wid * NCH

    def zfill(i, carry):
        def zcol(j, c2):
            zsl_v[i, pl.ds(j * 16, 16)] = jnp.zeros((16,), jnp.float32)
            return c2

        return lax.fori_loop(0, F // 16, zcol, carry)

    lax.fori_loop(0, BLK, zfill, 0)

    pltpu.sync_copy(src2_hbm.at[pl.ds(ch0, NCH)], si_v)
    pltpu.sync_copy(dst2_hbm.at[ch0], di_v.at[0])

    def zcopy(i, carry):
        row = (s + i * NS) * BLK
        pltpu.sync_copy(zsl_v, acc_sh.at[pl.ds(row, BLK)])
        return carry

    lax.fori_loop(0, _row_blocks(s), zcopy, 0)
    plsc.subcore_barrier()

    pltpu.async_copy(hs_hbm.at[si_v.at[0]], msg_v.at[0], gsem0)
    pltpu.async_copy(dst2_hbm.at[ch0 + 1], di_v.at[1], isem1)

    def body(g, carry):
        def step(p, gsem_p, gsem_o, isem_p):
            @pl.when(g + 1 < NCH)
            def _():
                pltpu.async_copy(hs_hbm.at[si_v.at[g + 1]], msg_v.at[1 - p], gsem_o)

            pltpu.make_async_copy(hs_hbm.at[si_v.at[g]], msg_v.at[p], gsem_p).wait()

            @pl.when(g >= 1)
            def _():
                pltpu.make_async_copy(dst2_hbm.at[ch0], di_v.at[p], isem_p).wait()

            pltpu.sync_copy(msg_v.at[p], acc_sh.at[di_v.at[p]], add=True)

            @pl.when(g + 2 < NCH)
            def _():
                pltpu.async_copy(dst2_hbm.at[ch0 + g + 2], di_v.at[p], isem_p)

        @pl.when(g % 2 == 0)
        def _():
            step(0, gsem0, gsem1, isem0)

        @pl.when(g % 2 == 1)
        def _():
            step(1, gsem1, gsem0, isem1)

        return carry

    lax.fori_loop(0, NCH, body, 0)
    plsc.subcore_barrier()

    def wcopy(i, carry):
        row = (s + i * NS) * BLK
        pltpu.sync_copy(acc_sh.at[pl.ds(row, BLK)], out_hbm.at[c, pl.ds(row, BLK)])
        return carry

    lax.fori_loop(0, _row_blocks(s), wcopy, 0)


def _norm_body(dp_ref, nm_ref):
    deg = dp_ref[0] + dp_ref[1]
    nm = lax.rsqrt(jnp.maximum(deg[:, :, 0:1], 1.0))
    row = lax.broadcasted_iota(jnp.int32, (1, N_PAD, 1), 1)
    nm_ref[...] = jnp.where(row < N, nm, 0.0)


def _scale_body(x_ref, ns_ref, hs_ref):
    hs_ref[...] = x_ref[...] * ns_ref[...]


def _bn_relu(p_ref, nd_ref, w_ref, b_ref, g_ref, be_ref):
    agg = (p_ref[0] + p_ref[1]) * nd_ref[...]
    hp = jnp.dot(agg, w_ref[...], preferred_element_type=jnp.float32) + b_ref[...]
    hpn = hp[0:N]
    mu = jnp.mean(hpn, axis=0, keepdims=True)
    var = jnp.mean((hpn - mu) ** 2, axis=0, keepdims=True)
    return jnp.maximum((hp - mu) * lax.rsqrt(var + 1e-5) * g_ref[...] + be_ref[...], 0.0)


def _dense_body(p_ref, nd_ref, ns_ref, w_ref, b_ref, g_ref, be_ref, o_ref):
    h = _bn_relu(p_ref, nd_ref, w_ref, b_ref, g_ref, be_ref)
    o_ref[...] = h * ns_ref[...]


def _head_body(p_ref, nd_ref, w_ref, b_ref, g_ref, be_ref, wc_ref, bc_ref, o_ref):
    h = _bn_relu(p_ref, nd_ref, w_ref, b_ref, g_ref, be_ref)
    hg = jnp.mean(h[0:N], axis=0, keepdims=True)
    o_ref[...] = jnp.dot(hg, wc_ref[...], preferred_element_type=jnp.float32) + bc_ref[...]


def kernel(x, edge_index, W1, b1, g1, be1, W2, b2, g2, be2, Wc, bc):
    f32 = jnp.float32
    i32 = jnp.int32
    pad = N + jnp.arange(E_PAD - E, dtype=i32) % (N_PAD - N)
    src2 = jnp.concatenate([edge_index[0].astype(i32), pad]).reshape(TOTAL_CHUNKS, CHUNK)
    dst2 = jnp.concatenate([edge_index[1].astype(i32), pad]).reshape(TOTAL_CHUNKS, CHUNK)
    x_pad = jnp.concatenate([x, jnp.zeros((N_PAD - N, F), f32)])

    dp = _sc_degrees(src2, dst2)

    nm = pl.pallas_call(
        _norm_body, out_shape=jax.ShapeDtypeStruct((2, N_PAD, 1), f32)
    )(dp)
    ns_col = nm[0]
    nd_col = nm[1]

    hs = pl.pallas_call(
        _scale_body, out_shape=jax.ShapeDtypeStruct((N_PAD, F), f32)
    )(x_pad, ns_col)

    p1 = _sc_aggregate(hs, src2, dst2)

    h1s = pl.pallas_call(
        _dense_body, out_shape=jax.ShapeDtypeStruct((N_PAD, F), f32)
    )(p1, nd_col, ns_col, W1, b1.reshape(1, F), g1.reshape(1, F), be1.reshape(1, F))

    p2 = _sc_aggregate(h1s, src2, dst2)

    out = pl.pallas_call(
        _head_body, out_shape=jax.ShapeDtypeStruct((1, 2), f32)
    )(p2, nd_col, W2, b2.reshape(1, F), g2.reshape(1, F), be2.reshape(1, F),
      Wc, bc.reshape(1, 2))
    return out


# speedup vs baseline: 3.9371x; 1.0269x over previous
import functools

import jax
import jax.numpy as jnp
from jax import lax
from jax.experimental import pallas as pl
from jax.experimental.pallas import tpu as pltpu
from jax.experimental.pallas import tpu_sc as plsc

N = 10000
F = 128
E = 320000
NC = 2
NS = 16
NW = NC * NS
CHUNK = 128
N_PAD = 10016
E_PAD = 327680
TOTAL_CHUNKS = E_PAD // CHUNK
NCH = TOTAL_CHUNKS // NW
BLK = 16
NBLK = N_PAD // BLK
BASE_BLKS = NBLK // NS
BLK_EXTRA = NBLK - BASE_BLKS * NS
DW = 16

_mesh = plsc.VectorSubcoreMesh(core_axis_name="c", subcore_axis_name="s")


def _row_blocks(s):
    return BASE_BLKS + jnp.where(s < BLK_EXTRA, 1, 0)


def _load_indices(src2_hbm, dst2_hbm, si_v, di_v, wid):
    ch0 = wid * NCH
    pltpu.sync_copy(src2_hbm.at[pl.ds(ch0, NCH)], si_v)
    pltpu.sync_copy(dst2_hbm.at[pl.ds(ch0, NCH)], di_v)


@functools.partial(
    pl.kernel,
    out_type=jax.ShapeDtypeStruct((NC, 2, N_PAD, DW), jnp.float32),
    mesh=_mesh,
    compiler_params=pltpu.CompilerParams(use_tc_tiling_on_sc=False),
    scratch_types=[
        pltpu.VMEM((NCH, CHUNK), jnp.int32),
        pltpu.VMEM((NCH, CHUNK), jnp.int32),
        pltpu.VMEM((CHUNK, DW), jnp.float32),
        pltpu.VMEM((BLK, DW), jnp.float32),
        pltpu.VMEM_SHARED((N_PAD, DW), jnp.float32),
        pltpu.VMEM_SHARED((N_PAD, DW), jnp.float32),
        pltpu.SemaphoreType.DMA,
    ],
)
def _sc_degrees(src2_hbm, dst2_hbm, out_hbm, si_v, di_v, ones_v, zsl_v,
                acca_sh, accb_sh, sem):
    c = lax.axis_index("c")
    s = lax.axis_index("s")
    wid = s * NC + c

    def fill(i, carry):
        ones_v[i, pl.ds(0, 16)] = jnp.full((16,), 1.0, jnp.float32)
        return carry

    lax.fori_loop(0, CHUNK, fill, 0)

    def zfill(i, carry):
        zsl_v[i, pl.ds(0, 16)] = jnp.zeros((16,), jnp.float32)
        return carry

    lax.fori_loop(0, BLK, zfill, 0)

    _load_indices(src2_hbm, dst2_hbm, si_v, di_v, wid)

    def zcopy(i, carry):
        row = (s + i * NS) * BLK
        pltpu.sync_copy(zsl_v, acca_sh.at[pl.ds(row, BLK)])
        pltpu.sync_copy(zsl_v, accb_sh.at[pl.ds(row, BLK)])
        return carry

    lax.fori_loop(0, _row_blocks(s), zcopy, 0)
    plsc.subcore_barrier()

    def body(g, carry):
        pltpu.async_copy(ones_v, acca_sh.at[si_v.at[g]], sem, add=True)
        pltpu.async_copy(ones_v, accb_sh.at[di_v.at[g]], sem, add=True)
        return carry

    lax.fori_loop(0, NCH, body, 0)

    def drain(i, carry):
        pltpu.make_async_copy(out_hbm.at[c, 0, pl.ds(0, CHUNK)], ones_v, sem).wait()
        return carry

    lax.fori_loop(0, 2 * NCH, drain, 0)
    plsc.subcore_barrier()

    def wcopy(i, carry):
        row = (s + i * NS) * BLK
        pltpu.sync_copy(acca_sh.at[pl.ds(row, BLK)], out_hbm.at[c, 0, pl.ds(row, BLK)])
        pltpu.sync_copy(accb_sh.at[pl.ds(row, BLK)], out_hbm.at[c, 1, pl.ds(row, BLK)])
        return carry

    lax.fori_loop(0, _row_blocks(s), wcopy, 0)


@functools.partial(
    pl.kernel,
    out_type=jax.ShapeDtypeStruct((NC, N_PAD, F), jnp.float32),
    mesh=_mesh,
    scratch_types=[
        pltpu.VMEM((NCH, CHUNK), jnp.int32),
        pltpu.VMEM((2, CHUNK), jnp.int32),
        pltpu.VMEM((2, CHUNK, F), jnp.float32),
        pltpu.VMEM((BLK, F), jnp.float32),
        pltpu.VMEM_SHARED((N_PAD, F), jnp.float32),
        pltpu.SemaphoreType.DMA,
        pltpu.SemaphoreType.DMA,
        pltpu.SemaphoreType.DMA,
        pltpu.SemaphoreType.DMA,
    ],
)
def _sc_aggregate(hs_hbm, src2_hbm, dst2_hbm, out_hbm, si_v, di_v, msg_v, zsl_v,
                  acc_sh, gsem0, gsem1, isem0, isem1):
    c = lax.axis_index("c")
    s = lax.axis_index("s")
    wid = s * NC + c
    ch0 = wid * NCH

    def zfill(i, carry):
        def zcol(j, c2):
            zsl_v[i, pl.ds(j * 16, 16)] = jnp.zeros((16,), jnp.float32)
            return c2

        return lax.fori_loop(0, F // 16, zcol, carry)

    lax.fori_loop(0, BLK, zfill, 0)

    pltpu.sync_copy(src2_hbm.at[pl.ds(ch0, NCH)], si_v)
    pltpu.sync_copy(dst2_hbm.at[ch0], di_v.at[0])

    def zcopy(i, carry):
        row = (s + i * NS) * BLK
        pltpu.sync_copy(zsl_v, acc_sh.at[pl.ds(row, BLK)])
        return carry

    lax.fori_loop(0, _row_blocks(s), zcopy, 0)
    plsc.subcore_barrier()

    pltpu.async_copy(hs_hbm.at[si_v.at[0]], msg_v.at[0], gsem0)
    pltpu.async_copy(dst2_hbm.at[ch0 + 1], di_v.at[1], isem1)

    def body(g, carry):
        def step(p, gsem_p, gsem_o, isem_p):
            @pl.when(g + 1 < NCH)
            def _():
                pltpu.async_copy(hs_hbm.at[si_v.at[g + 1]], msg_v.at[1 - p], gsem_o)

            pltpu.make_async_copy(hs_hbm.at[si_v.at[g]], msg_v.at[p], gsem_p).wait()

            @pl.when(g >= 1)
            def _():
                pltpu.make_async_copy(dst2_hbm.at[ch0], di_v.at[p], isem_p).wait()

            pltpu.sync_copy(msg_v.at[p], acc_sh.at[di_v.at[p]], add=True)

            @pl.when(g + 2 < NCH)
            def _():
                pltpu.async_copy(dst2_hbm.at[ch0 + g + 2], di_v.at[p], isem_p)

        @pl.when(g % 2 == 0)
        def _():
            step(0, gsem0, gsem1, isem0)

        @pl.when(g % 2 == 1)
        def _():
            step(1, gsem1, gsem0, isem1)

        return carry

    lax.fori_loop(0, NCH, body, 0)
    plsc.subcore_barrier()

    def wcopy(i, carry):
        row = (s + i * NS) * BLK
        pltpu.sync_copy(acc_sh.at[pl.ds(row, BLK)], out_hbm.at[c, pl.ds(row, BLK)])
        return carry

    lax.fori_loop(0, _row_blocks(s), wcopy, 0)


def _prep_body(dp_ref, x_ref, hs_ref, nm_ref):
    deg = dp_ref[0] + dp_ref[1]
    nm = lax.rsqrt(jnp.maximum(deg[:, :, 0:1], 1.0))
    row = lax.broadcasted_iota(jnp.int32, (1, N_PAD, 1), 1)
    nm = jnp.where(row < N, nm, 0.0)
    nm_ref[...] = nm
    hs_ref[...] = x_ref[...] * nm[0]


def _bn_relu(p_ref, nd_ref, w_ref, b_ref, g_ref, be_ref):
    agg = (p_ref[0] + p_ref[1]) * nd_ref[...]
    hp = jnp.dot(agg, w_ref[...], preferred_element_type=jnp.float32) + b_ref[...]
    hpn = hp[0:N]
    mu = jnp.mean(hpn, axis=0, keepdims=True)
    var = jnp.mean((hpn - mu) ** 2, axis=0, keepdims=True)
    return jnp.maximum((hp - mu) * lax.rsqrt(var + 1e-5) * g_ref[...] + be_ref[...], 0.0)


def _dense_body(p_ref, nd_ref, ns_ref, w_ref, b_ref, g_ref, be_ref, o_ref):
    h = _bn_relu(p_ref, nd_ref, w_ref, b_ref, g_ref, be_ref)
    o_ref[...] = h * ns_ref[...]


def _head_body(p_ref, nd_ref, w_ref, b_ref, g_ref, be_ref, wc_ref, bc_ref, o_ref):
    h = _bn_relu(p_ref, nd_ref, w_ref, b_ref, g_ref, be_ref)
    hg = jnp.mean(h[0:N], axis=0, keepdims=True)
    o_ref[...] = jnp.dot(hg, wc_ref[...], preferred_element_type=jnp.float32) + bc_ref[...]


def kernel(x, edge_index, W1, b1, g1, be1, W2, b2, g2, be2, Wc, bc):
    f32 = jnp.float32
    i32 = jnp.int32
    pad = N + jnp.arange(E_PAD - E, dtype=i32) % (N_PAD - N)
    src2 = jnp.concatenate([edge_index[0].astype(i32), pad]).reshape(TOTAL_CHUNKS, CHUNK)
    dst2 = jnp.concatenate([edge_index[1].astype(i32), pad]).reshape(TOTAL_CHUNKS, CHUNK)
    x_pad = jnp.concatenate([x, jnp.zeros((N_PAD - N, F), f32)])

    dp = _sc_degrees(src2, dst2)

    hs, nm = pl.pallas_call(
        _prep_body,
        out_shape=[jax.ShapeDtypeStruct((N_PAD, F), f32),
                   jax.ShapeDtypeStruct((2, N_PAD, 1), f32)],
    )(dp, x_pad)
    ns_col = nm[0]
    nd_col = nm[1]

    p1 = _sc_aggregate(hs, src2, dst2)

    h1s = pl.pallas_call(
        _dense_body, out_shape=jax.ShapeDtypeStruct((N_PAD, F), f32)
    )(p1, nd_col, ns_col, W1, b1.reshape(1, F), g1.reshape(1, F), be1.reshape(1, F))

    p2 = _sc_aggregate(h1s, src2, dst2)

    out = pl.pallas_call(
        _head_body, out_shape=jax.ShapeDtypeStruct((1, 2), f32)
    )(p2, nd_col, W2, b2.reshape(1, F), g2.reshape(1, F), be2.reshape(1, F),
      Wc, bc.reshape(1, 2))
    return out
